# Initial kernel scaffold; baseline (speedup 1.0000x reference)
import jax
import jax.numpy as jnp
from jax.experimental import pallas as pl


def kernel(x, edge_index, edge_weight, W):
    raise NotImplementedError("write your pallas kernel here")


# speedup vs baseline: 2.6692x; 2.6692x over previous
import functools

import jax
import jax.numpy as jnp
from jax import lax
from jax.experimental import pallas as pl
from jax.experimental.pallas import tpu as pltpu
from jax.experimental.pallas import tpu_sc as plsc

_NC = 2
_NS = 16
_LANES = 16
_B = 128


def _cheb_power_coeffs(K):
    c = [[1.0] + [0.0] * (K - 1)]
    if K > 1:
        c.append([-1.0, 1.0] + [0.0] * (K - 2))
    for k in range(2, K):
        shifted = [0.0] + c[k - 1][: K - 1]
        c.append([
            2.0 * (shifted[j] - c[k - 1][j]) - c[k - 2][j] for j in range(K)
        ])
    return c


def _make_spmm(n, m, f, e_pad):
    ept = e_pad // _NS
    nbatch = ept // _B
    rstep = 8 * (m // (8 * _NS))
    rlen = m - (_NS - 1) * rstep
    npc = n // _NC

    mesh = plsc.VectorSubcoreMesh(core_axis_name="c", subcore_axis_name="s")

    @functools.partial(
        pl.kernel,
        mesh=mesh,
        out_type=jax.ShapeDtypeStruct((n, m, f), jnp.float32),
        scratch_types=[
            pltpu.VMEM_SHARED((m, f), jnp.float32),
            pltpu.VMEM((_B,), jnp.int32),
            pltpu.VMEM((_B,), jnp.int32),
            pltpu.VMEM((_B,), jnp.float32),
            pltpu.VMEM((_B, f), jnp.float32),
            pltpu.SemaphoreType.DMA,
        ],
    )
    def spmm(sprev, srcr, dstr, wr, zr, snext, acc, idxs, idxd, ws, rows, sem):
        cid = lax.axis_index("c")
        sid = lax.axis_index("s")
        row0 = sid * rstep

        for i in range(npc):
            nn = cid * npc + i
            off = nn * m
            pltpu.sync_copy(zr.at[pl.ds(row0, rlen)], acc.at[pl.ds(row0, rlen)])
            plsc.subcore_barrier()

            def batch_body(b, _):
                base = sid * ept + b * _B
                pltpu.sync_copy(srcr.at[pl.ds(base, _B)], idxs)
                pltpu.sync_copy(dstr.at[pl.ds(base, _B)], idxd)
                pltpu.sync_copy(wr.at[pl.ds(base, _B)], ws)
                for v in range(_B // _LANES):
                    sl = pl.ds(v * _LANES, _LANES)
                    idxs[sl] = idxs[sl] + off
                pltpu.async_copy(sprev.at[idxs], rows, sem).wait()

                def scale_group(g, carry):
                    wv16 = ws[pl.ds(g * _LANES, _LANES)]
                    for l in range(_LANES):
                        r = g * _LANES + l
                        wv = wv16[l]
                        for cchunk in range(f // _LANES):
                            csl = pl.ds(cchunk * _LANES, _LANES)
                            rows[r, csl] = rows[r, csl] * wv
                    return carry

                lax.fori_loop(0, _B // _LANES, scale_group, 0)
                pltpu.sync_copy(rows, acc.at[idxd], add=True)
                return _

            lax.fori_loop(0, nbatch, batch_body, 0)
            plsc.subcore_barrier()
            pltpu.sync_copy(
                acc.at[pl.ds(row0, rlen)],
                snext.at[nn].at[pl.ds(row0, rlen)],
            )
            plsc.subcore_barrier()

    return spmm


def _make_combine(n, m, f, fout, K, bm):
    def body(*refs):
        s_refs = refs[:K]
        w_ref = refs[K]
        o_ref = refs[K + 1]
        acc = jnp.dot(
            s_refs[0][0], w_ref[0], preferred_element_type=jnp.float32
        )
        for j in range(1, K):
            acc = acc + jnp.dot(
                s_refs[j][0], w_ref[j], preferred_element_type=jnp.float32
            )
        o_ref[0] = acc

    s_spec = pl.BlockSpec((1, bm, f), lambda nn, mi: (nn, mi, 0))
    return pl.pallas_call(
        body,
        grid=(n, m // bm),
        in_specs=[s_spec] * K + [
            pl.BlockSpec((K, f, fout), lambda nn, mi: (0, 0, 0))
        ],
        out_specs=pl.BlockSpec((1, bm, fout), lambda nn, mi: (nn, mi, 0)),
        out_shape=jax.ShapeDtypeStruct((n, m, fout), jnp.float32),
    )


def kernel(x, edge_index, edge_weight, W):
    n, m, fin = x.shape
    fout = W.shape[1]
    K = W.shape[0] // fin

    C = jnp.asarray(_cheb_power_coeffs(K), dtype=jnp.float32)
    Wk = W.reshape(fin, K, fout)
    Wp = jnp.einsum("kj,fko->jfo", C, Wk)

    granule = _NS * _B
    e = edge_index.shape[1]
    e_pad = ((e + granule - 1) // granule) * granule
    pad = e_pad - e
    src = jnp.pad(edge_index[0], (0, pad))
    dst = jnp.pad(edge_index[1], (0, pad))
    w = jnp.pad(edge_weight, (0, pad))
    zeros = jnp.zeros((m, fin), jnp.float32)

    spmm = _make_spmm(n, m, fin, e_pad)
    powers = [x]
    for _ in range(K - 1):
        prev_flat = powers[-1].reshape(n * m, fin)
        powers.append(spmm(prev_flat, src, dst, w, zeros))

    bm = 1000 if m % 1000 == 0 else 8
    combine = _make_combine(n, m, fin, fout, K, bm)
    return combine(*powers, Wp)
